# Initial kernel scaffold; baseline (speedup 1.0000x reference)
import jax
import jax.numpy as jnp
from jax.experimental import pallas as pl


def kernel(x, gate_w, moe_bias, sh_up_w, sh_up_b, sh_down_w, sh_down_b, ex_up_w, ex_up_b, ex_down_w, ex_down_b):
    raise NotImplementedError("write your pallas kernel here")


# speedup vs baseline: 2.9872x; 2.9872x over previous
import functools

import jax
import jax.numpy as jnp
from jax import lax
from jax.experimental import pallas as pl
from jax.experimental.pallas import tpu as pltpu
from jax.experimental.pallas import tpu_sc as plsc

H = 1024
I = 4096
E = 8
K = 2
BT = 256
TOK = 512
LANES = 128
NW = 32
GCH = 64


def _gelu(v):
    return 0.5 * v * (1.0 + lax.erf(v * 0.7071067811865476))


def _router_body(x_ref, gw_ref, gb_ref, out_ref, cnt_ref, base_ref):
    t = pl.program_id(0)

    @pl.when(t == 0)
    def _():
        base_ref[...] = jnp.zeros_like(base_ref)

    xb = x_ref[...]
    logits = lax.dot_general(xb, gw_ref[...], (((1,), (1,)), ((), ())),
                             preferred_element_type=jnp.float32)
    lane = lax.broadcasted_iota(jnp.int32, logits.shape, 1)
    valid = lane < E
    sig = jax.nn.sigmoid(logits + gb_ref[...])
    s = jnp.where(valid, sig, -1.0)
    max0 = jnp.max(s, axis=1, keepdims=True)
    idx0 = jnp.min(jnp.where((s == max0) & valid, lane, LANES), axis=1,
                   keepdims=True)
    s1 = jnp.where(lane == idx0, -1.0, s)
    max1 = jnp.max(s1, axis=1, keepdims=True)
    idx1 = jnp.min(jnp.where((s1 == max1) & valid & (lane != idx0), lane,
                             LANES), axis=1, keepdims=True)
    oh0 = (lane == idx0).astype(jnp.float32)
    oh1 = (lane == idx1).astype(jnp.float32)
    m = oh0 + oh1
    r_i = lax.broadcasted_iota(jnp.int32, (TOK, TOK), 0)
    c_i = lax.broadcasted_iota(jnp.int32, (TOK, TOK), 1)
    tril = (r_i > c_i).astype(jnp.float32)
    excl = lax.dot_general(tril, m, (((1,), (0,)), ((), ())),
                           preferred_element_type=jnp.float32)
    base = base_ref[...]
    rank0 = jnp.sum(oh0 * (excl + base), axis=1, keepdims=True)
    rank1 = jnp.sum(oh1 * (excl + base), axis=1, keepdims=True)
    newbase = base + jnp.sum(m, axis=0, keepdims=True)
    base_ref[...] = newbase
    cnt_ref[...] = jnp.broadcast_to(newbase, cnt_ref.shape)
    denom = max0 + max1 + 1e-6
    w0 = max0 / denom
    w1 = max1 / denom
    out_ref[...] = (jnp.where(lane == 0, w0, 0.0)
                    + jnp.where(lane == 1, w1, 0.0)
                    + jnp.where(lane == 2, idx0.astype(jnp.float32), 0.0)
                    + jnp.where(lane == 3, idx1.astype(jnp.float32), 0.0)
                    + jnp.where(lane == 4, rank0, 0.0)
                    + jnp.where(lane == 5, rank1, 0.0))


def _router(x2d, gw_pad, gb_pad, T):
    return pl.pallas_call(
        _router_body,
        grid=(T // TOK,),
        in_specs=[
            pl.BlockSpec((TOK, H), lambda t: (t, 0)),
            pl.BlockSpec((LANES, H), lambda t: (0, 0)),
            pl.BlockSpec((1, LANES), lambda t: (0, 0)),
        ],
        out_specs=[
            pl.BlockSpec((TOK, LANES), lambda t: (t, 0)),
            pl.BlockSpec((8, LANES), lambda t: (0, 0)),
        ],
        out_shape=[
            jax.ShapeDtypeStruct((T, LANES), jnp.float32),
            jax.ShapeDtypeStruct((8, LANES), jnp.float32),
        ],
        scratch_shapes=[pltpu.VMEM((1, LANES), jnp.float32)],
    )(x2d, gw_pad, gb_pad)


def _make_g1(T, P):
    mesh = plsc.VectorSubcoreMesh(core_axis_name="c", subcore_axis_name="s")
    rows_w = P // NW
    pairs = T * K

    @functools.partial(
        pl.kernel,
        out_type=jax.ShapeDtypeStruct((P, H), jnp.float32),
        mesh=mesh,
        compiler_params=pltpu.CompilerParams(needs_layout_passes=False),
        scratch_types=[
            pltpu.VMEM((16,), jnp.int32),
            pltpu.VMEM((pairs,), jnp.int32),
            pltpu.VMEM((pairs,), jnp.int32),
            pltpu.VMEM((P,), jnp.int32),
            pltpu.VMEM((GCH, H), jnp.float32),
            pltpu.SemaphoreType.DMA,
        ],
    )
    def g1(po_hbm, ef_hbm, rf_hbm, x_hbm, xs_hbm, po_v, ef_v, rf_v, rid_v,
           rows_v, sem):
        cid = lax.axis_index("c")
        sid = lax.axis_index("s")
        wid = sid * 2 + cid
        pltpu.sync_copy(po_hbm, po_v)
        pltpu.sync_copy(ef_hbm, ef_v)
        pltpu.sync_copy(rf_hbm, rf_v)

        def zero(j, carry):
            rid_v[pl.ds(j * 16, 16)] = jnp.zeros((16,), jnp.int32)
            return carry

        lax.fori_loop(0, P // 16, zero, 0)

        def scat(j, carry):
            e = ef_v[pl.ds(j * 16, 16)]
            r = rf_v[pl.ds(j * 16, 16)]
            off = plsc.load_gather(po_v, [e])
            pos = off + r
            tok = lax.shift_right_logical(
                lax.broadcasted_iota(jnp.int32, (16,), 0) + j * 16, 1)
            plsc.store_scatter(rid_v, [pos], tok)
            return carry

        lax.fori_loop(0, pairs // 16, scat, 0)

        base = wid * rows_w
        for ch in range(rows_w // GCH):
            b = base + ch * GCH
            pltpu.async_copy(x_hbm.at[rid_v.at[pl.ds(b, GCH)]], rows_v,
                             sem).wait()
            pltpu.sync_copy(rows_v, xs_hbm.at[pl.ds(b, GCH)])

    return g1


def _expert_body(te_ref, xs_ref, uw_ref, ub_ref, dw_ref, db_ref, ys_ref):
    xb = xs_ref[...].astype(jnp.bfloat16)
    h = lax.dot_general(xb, uw_ref[0], (((1,), (1,)), ((), ())),
                        preferred_element_type=jnp.float32) + ub_ref[0]
    g = _gelu(h).astype(jnp.bfloat16)
    ys_ref[...] = lax.dot_general(g, dw_ref[0], (((1,), (1,)), ((), ())),
                                  preferred_element_type=jnp.float32) + db_ref[0]


def _expert_ffn(xs, te, ex_up_w, ex_up_b, ex_down_w, ex_down_b, P):
    NT = P // BT
    grid_spec = pltpu.PrefetchScalarGridSpec(
        num_scalar_prefetch=1,
        grid=(NT,),
        in_specs=[
            pl.BlockSpec((BT, H), lambda t, te: (t, 0)),
            pl.BlockSpec((1, I, H), lambda t, te: (te[t], 0, 0)),
            pl.BlockSpec((1, 1, I), lambda t, te: (te[t], 0, 0)),
            pl.BlockSpec((1, H, I), lambda t, te: (te[t], 0, 0)),
            pl.BlockSpec((1, 1, H), lambda t, te: (te[t], 0, 0)),
        ],
        out_specs=pl.BlockSpec((BT, H), lambda t, te: (t, 0)),
    )
    return pl.pallas_call(
        _expert_body,
        grid_spec=grid_spec,
        out_shape=jax.ShapeDtypeStruct((P, H), jnp.float32),
    )(te, xs, ex_up_w, ex_up_b.reshape(E, 1, I), ex_down_w,
      ex_down_b.reshape(E, 1, H))


def _make_g2(T, P):
    mesh = plsc.VectorSubcoreMesh(core_axis_name="c", subcore_axis_name="s")
    tok_w = T // NW

    @functools.partial(
        pl.kernel,
        out_type=(jax.ShapeDtypeStruct((T, H), jnp.float32),
                  jax.ShapeDtypeStruct((T, H), jnp.float32)),
        mesh=mesh,
        compiler_params=pltpu.CompilerParams(needs_layout_passes=False),
        scratch_types=[
            pltpu.VMEM((16,), jnp.int32),
            pltpu.VMEM((tok_w,), jnp.int32),
            pltpu.VMEM((tok_w,), jnp.int32),
            pltpu.VMEM((tok_w,), jnp.int32),
            pltpu.VMEM((GCH, H), jnp.float32),
            pltpu.SemaphoreType.DMA,
        ],
    )
    def g2(po_hbm, e0_hbm, r0_hbm, e1_hbm, r1_hbm, ys_hbm, a_hbm, b_hbm,
           po_v, e_v, r_v, idx_v, rows_v, sem):
        cid = lax.axis_index("c")
        sid = lax.axis_index("s")
        wid = sid * 2 + cid
        base = wid * tok_w
        pltpu.sync_copy(po_hbm, po_v)
        for e_hbm, r_hbm, out_hbm in ((e0_hbm, r0_hbm, a_hbm),
                                      (e1_hbm, r1_hbm, b_hbm)):
            pltpu.sync_copy(e_hbm.at[pl.ds(base, tok_w)], e_v)
            pltpu.sync_copy(r_hbm.at[pl.ds(base, tok_w)], r_v)

            def mkpos(j, carry):
                e = e_v[pl.ds(j * 16, 16)]
                r = r_v[pl.ds(j * 16, 16)]
                idx_v[pl.ds(j * 16, 16)] = plsc.load_gather(po_v, [e]) + r
                return carry

            lax.fori_loop(0, tok_w // 16, mkpos, 0)
            for ch in range(tok_w // GCH):
                pltpu.async_copy(ys_hbm.at[idx_v.at[pl.ds(ch * GCH, GCH)]],
                                 rows_v, sem).wait()
                pltpu.sync_copy(rows_v,
                                out_hbm.at[pl.ds(base + ch * GCH, GCH)])

    return g2


def _finish_body(x_ref, pk_ref, a_ref, b_ref, uw_ref, ub_ref, dw_ref, db_ref,
                 out_ref):
    xb = x_ref[...].astype(jnp.bfloat16)
    h = lax.dot_general(xb, uw_ref[...], (((1,), (1,)), ((), ())),
                        preferred_element_type=jnp.float32) + ub_ref[...]
    g = _gelu(h).astype(jnp.bfloat16)
    sh = lax.dot_general(g, dw_ref[...], (((1,), (1,)), ((), ())),
                         preferred_element_type=jnp.float32) + db_ref[...]
    pk = pk_ref[...]
    lane = lax.broadcasted_iota(jnp.int32, pk.shape, 1)
    w0 = jnp.sum(jnp.where(lane == 0, pk, 0.0), axis=1, keepdims=True)
    w1 = jnp.sum(jnp.where(lane == 1, pk, 0.0), axis=1, keepdims=True)
    t = sh + w0 * a_ref[...] + w1 * b_ref[...]
    mo = 0.1 * t
    out_ref[...] = mo / (jnp.max(jnp.abs(mo), axis=1, keepdims=True) + 1e-6)


def _finish(x2d, packed, a, b, sh_up_w, sh_up_b, sh_down_w, sh_down_b, T):
    return pl.pallas_call(
        _finish_body,
        grid=(T // TOK,),
        in_specs=[
            pl.BlockSpec((TOK, H), lambda t: (t, 0)),
            pl.BlockSpec((TOK, LANES), lambda t: (t, 0)),
            pl.BlockSpec((TOK, H), lambda t: (t, 0)),
            pl.BlockSpec((TOK, H), lambda t: (t, 0)),
            pl.BlockSpec((I, H), lambda t: (0, 0)),
            pl.BlockSpec((1, I), lambda t: (0, 0)),
            pl.BlockSpec((H, I), lambda t: (0, 0)),
            pl.BlockSpec((1, H), lambda t: (0, 0)),
        ],
        out_specs=pl.BlockSpec((TOK, H), lambda t: (t, 0)),
        out_shape=jax.ShapeDtypeStruct((T, H), jnp.float32),
    )(x2d, packed, a, b, sh_up_w, sh_up_b, sh_down_w, sh_down_b)


def kernel(x, gate_w, moe_bias, sh_up_w, sh_up_b, sh_down_w, sh_down_b,
           ex_up_w, ex_up_b, ex_down_w, ex_down_b):
    B, S, _ = x.shape
    T = B * S
    P = ((T * K + E * (BT - 1)) + BT - 1) // BT * BT
    x2d = x.reshape(T, H)

    gw_pad = jnp.zeros((LANES, H), jnp.float32).at[:E].set(gate_w)
    gb_pad = jnp.zeros((1, LANES), jnp.float32).at[0, :E].set(moe_bias)

    packed, cnt = _router(x2d, gw_pad, gb_pad, T)
    counts = cnt[0, :E].astype(jnp.int32)
    pad_cnt = ((counts + BT - 1) // BT) * BT
    pad_end = jnp.cumsum(pad_cnt)
    pad_off = (pad_end - pad_cnt).astype(jnp.int32)
    pad_off16 = jnp.zeros((16,), jnp.int32).at[:E].set(pad_off)

    tstart = jnp.arange(P // BT, dtype=jnp.int32) * BT
    te = jnp.minimum(jnp.searchsorted(pad_end, tstart, side="right"),
                     E - 1).astype(jnp.int32)

    e0 = packed[:, 2].astype(jnp.int32)
    e1 = packed[:, 3].astype(jnp.int32)
    r0 = packed[:, 4].astype(jnp.int32)
    r1 = packed[:, 5].astype(jnp.int32)
    ef = jnp.stack([e0, e1], axis=1).reshape(-1)
    rf = jnp.stack([r0, r1], axis=1).reshape(-1)

    xs = _make_g1(T, P)(pad_off16, ef, rf, x2d)
    ys = _expert_ffn(xs, te, ex_up_w.astype(jnp.bfloat16), ex_up_b,
                     ex_down_w.astype(jnp.bfloat16), ex_down_b, P)
    a, b = _make_g2(T, P)(pad_off16, e0, r0, e1, r1, ys)
    out = _finish(x2d, packed, a, b, sh_up_w.astype(jnp.bfloat16),
                  sh_up_b.reshape(1, I), sh_down_w.astype(jnp.bfloat16),
                  sh_down_b.reshape(1, H), T)
    return out.reshape(B, S, H)


# speedup vs baseline: 3.5712x; 1.1955x over previous
import functools

import jax
import jax.numpy as jnp
from jax import lax
from jax.experimental import pallas as pl
from jax.experimental.pallas import tpu as pltpu
from jax.experimental.pallas import tpu_sc as plsc

H = 1024
I = 4096
E = 8
K = 2
BT = 256
TOK = 512
LANES = 128
NW = 32
GCH = 64


def _gelu(v):
    return 0.5 * v * (1.0 + lax.erf(v * 0.7071067811865476))


def _router_body(x_ref, gw_ref, gb_ref, out_ref, cnt_ref, base_ref):
    t = pl.program_id(0)

    @pl.when(t == 0)
    def _():
        base_ref[...] = jnp.zeros_like(base_ref)

    xb = x_ref[...]
    logits = lax.dot_general(xb, gw_ref[...], (((1,), (1,)), ((), ())),
                             preferred_element_type=jnp.float32)
    lane = lax.broadcasted_iota(jnp.int32, logits.shape, 1)
    valid = lane < E
    sig = jax.nn.sigmoid(logits + gb_ref[...])
    s = jnp.where(valid, sig, -1.0)
    max0 = jnp.max(s, axis=1, keepdims=True)
    idx0 = jnp.min(jnp.where((s == max0) & valid, lane, LANES), axis=1,
                   keepdims=True)
    s1 = jnp.where(lane == idx0, -1.0, s)
    max1 = jnp.max(s1, axis=1, keepdims=True)
    idx1 = jnp.min(jnp.where((s1 == max1) & valid & (lane != idx0), lane,
                             LANES), axis=1, keepdims=True)
    oh0 = (lane == idx0).astype(jnp.float32)
    oh1 = (lane == idx1).astype(jnp.float32)
    m = oh0 + oh1
    r_i = lax.broadcasted_iota(jnp.int32, (TOK, TOK), 0)
    c_i = lax.broadcasted_iota(jnp.int32, (TOK, TOK), 1)
    tril = (r_i > c_i).astype(jnp.float32)
    excl = lax.dot_general(tril, m, (((1,), (0,)), ((), ())),
                           preferred_element_type=jnp.float32)
    base = base_ref[...]
    rank0 = jnp.sum(oh0 * (excl + base), axis=1, keepdims=True)
    rank1 = jnp.sum(oh1 * (excl + base), axis=1, keepdims=True)
    newbase = base + jnp.sum(m, axis=0, keepdims=True)
    base_ref[...] = newbase
    cnt_ref[...] = jnp.broadcast_to(newbase, cnt_ref.shape)
    denom = max0 + max1 + 1e-6
    w0 = max0 / denom
    w1 = max1 / denom
    out_ref[...] = (jnp.where(lane == 0, w0, 0.0)
                    + jnp.where(lane == 1, w1, 0.0)
                    + jnp.where(lane == 2, idx0.astype(jnp.float32), 0.0)
                    + jnp.where(lane == 3, idx1.astype(jnp.float32), 0.0)
                    + jnp.where(lane == 4, rank0, 0.0)
                    + jnp.where(lane == 5, rank1, 0.0))


def _router(x2d, gw_pad, gb_pad, T):
    return pl.pallas_call(
        _router_body,
        grid=(T // TOK,),
        in_specs=[
            pl.BlockSpec((TOK, H), lambda t: (t, 0)),
            pl.BlockSpec((LANES, H), lambda t: (0, 0)),
            pl.BlockSpec((1, LANES), lambda t: (0, 0)),
        ],
        out_specs=[
            pl.BlockSpec((TOK, LANES), lambda t: (t, 0)),
            pl.BlockSpec((8, LANES), lambda t: (0, 0)),
        ],
        out_shape=[
            jax.ShapeDtypeStruct((T, LANES), jnp.float32),
            jax.ShapeDtypeStruct((8, LANES), jnp.float32),
        ],
        scratch_shapes=[pltpu.VMEM((1, LANES), jnp.float32)],
    )(x2d, gw_pad, gb_pad)


def _make_g1(T, P):
    mesh = plsc.VectorSubcoreMesh(core_axis_name="c", subcore_axis_name="s")
    tok_w = T // NW
    CH = 32
    NCH = tok_w // CH

    @functools.partial(
        pl.kernel,
        out_type=jax.ShapeDtypeStruct((P, H), jnp.float32),
        mesh=mesh,
        compiler_params=pltpu.CompilerParams(needs_layout_passes=False),
        scratch_types=[
            pltpu.VMEM((16,), jnp.int32),
            pltpu.VMEM((tok_w,), jnp.int32),
            pltpu.VMEM((tok_w,), jnp.int32),
            pltpu.VMEM((tok_w,), jnp.int32),
            pltpu.VMEM((tok_w,), jnp.int32),
            pltpu.VMEM((2 * NCH, CH), jnp.int32),
            pltpu.VMEM((2, CH, H), jnp.float32),
            pltpu.SemaphoreType.DMA,
        ],
    )
    def g1(po_hbm, e0_hbm, r0_hbm, e1_hbm, r1_hbm, x_hbm, xs_hbm,
           po_v, e0_v, r0_v, e1_v, r1_v, idx_v, rows_v, sem):
        cid = lax.axis_index("c")
        sid = lax.axis_index("s")
        wid = sid * 2 + cid
        tb = wid * tok_w
        pltpu.sync_copy(po_hbm, po_v)
        pltpu.sync_copy(e0_hbm.at[pl.ds(tb, tok_w)], e0_v)
        pltpu.sync_copy(r0_hbm.at[pl.ds(tb, tok_w)], r0_v)
        pltpu.sync_copy(e1_hbm.at[pl.ds(tb, tok_w)], e1_v)
        pltpu.sync_copy(r1_hbm.at[pl.ds(tb, tok_w)], r1_v)
        for c in range(NCH):
            for m in range(CH // 16):
                sl = pl.ds(c * CH + m * 16, 16)
                idx_v[c, pl.ds(m * 16, 16)] = (
                    plsc.load_gather(po_v, [e0_v[sl]]) + r0_v[sl])
                idx_v[NCH + c, pl.ds(m * 16, 16)] = (
                    plsc.load_gather(po_v, [e1_v[sl]]) + r1_v[sl])
        pend = [None, None]
        for c in range(NCH):
            b = c % 2
            if pend[b] is not None:
                pend[b][0].wait()
                pend[b][1].wait()
            pltpu.sync_copy(x_hbm.at[pl.ds(tb + c * CH, CH)], rows_v.at[b])
            d0 = pltpu.async_copy(rows_v.at[b], xs_hbm.at[idx_v.at[c]], sem)
            d1 = pltpu.async_copy(rows_v.at[b], xs_hbm.at[idx_v.at[NCH + c]],
                                  sem)
            pend[b] = (d0, d1)
        for b in range(2):
            if pend[b] is not None:
                pend[b][0].wait()
                pend[b][1].wait()

    return g1


def _expert_body(te_ref, xs_ref, uw_ref, ub_ref, dw_ref, db_ref, ys_ref):
    xb = xs_ref[...].astype(jnp.bfloat16)
    h = lax.dot_general(xb, uw_ref[0], (((1,), (1,)), ((), ())),
                        preferred_element_type=jnp.float32) + ub_ref[0]
    g = _gelu(h).astype(jnp.bfloat16)
    ys_ref[...] = lax.dot_general(g, dw_ref[0], (((1,), (1,)), ((), ())),
                                  preferred_element_type=jnp.float32) + db_ref[0]


def _expert_ffn(xs, te, ex_up_w, ex_up_b, ex_down_w, ex_down_b, P):
    NT = P // BT
    grid_spec = pltpu.PrefetchScalarGridSpec(
        num_scalar_prefetch=1,
        grid=(NT,),
        in_specs=[
            pl.BlockSpec((BT, H), lambda t, te: (t, 0)),
            pl.BlockSpec((1, I, H), lambda t, te: (te[t], 0, 0)),
            pl.BlockSpec((1, 1, I), lambda t, te: (te[t], 0, 0)),
            pl.BlockSpec((1, H, I), lambda t, te: (te[t], 0, 0)),
            pl.BlockSpec((1, 1, H), lambda t, te: (te[t], 0, 0)),
        ],
        out_specs=pl.BlockSpec((BT, H), lambda t, te: (t, 0)),
    )
    return pl.pallas_call(
        _expert_body,
        grid_spec=grid_spec,
        out_shape=jax.ShapeDtypeStruct((P, H), jnp.float32),
    )(te, xs, ex_up_w, ex_up_b.reshape(E, 1, I), ex_down_w,
      ex_down_b.reshape(E, 1, H))


def _make_g2(T, P):
    mesh = plsc.VectorSubcoreMesh(core_axis_name="c", subcore_axis_name="s")
    tok_w = T // NW

    @functools.partial(
        pl.kernel,
        out_type=(jax.ShapeDtypeStruct((T, H), jnp.float32),
                  jax.ShapeDtypeStruct((T, H), jnp.float32)),
        mesh=mesh,
        compiler_params=pltpu.CompilerParams(needs_layout_passes=False),
        scratch_types=[
            pltpu.VMEM((16,), jnp.int32),
            pltpu.VMEM((tok_w,), jnp.int32),
            pltpu.VMEM((tok_w,), jnp.int32),
            pltpu.VMEM((tok_w,), jnp.int32),
            pltpu.VMEM((GCH, H), jnp.float32),
            pltpu.SemaphoreType.DMA,
        ],
    )
    def g2(po_hbm, e0_hbm, r0_hbm, e1_hbm, r1_hbm, ys_hbm, a_hbm, b_hbm,
           po_v, e_v, r_v, idx_v, rows_v, sem):
        cid = lax.axis_index("c")
        sid = lax.axis_index("s")
        wid = sid * 2 + cid
        base = wid * tok_w
        pltpu.sync_copy(po_hbm, po_v)
        for e_hbm, r_hbm, out_hbm in ((e0_hbm, r0_hbm, a_hbm),
                                      (e1_hbm, r1_hbm, b_hbm)):
            pltpu.sync_copy(e_hbm.at[pl.ds(base, tok_w)], e_v)
            pltpu.sync_copy(r_hbm.at[pl.ds(base, tok_w)], r_v)

            def mkpos(j, carry):
                e = e_v[pl.ds(j * 16, 16)]
                r = r_v[pl.ds(j * 16, 16)]
                idx_v[pl.ds(j * 16, 16)] = plsc.load_gather(po_v, [e]) + r
                return carry

            lax.fori_loop(0, tok_w // 16, mkpos, 0)
            for ch in range(tok_w // GCH):
                pltpu.async_copy(ys_hbm.at[idx_v.at[pl.ds(ch * GCH, GCH)]],
                                 rows_v, sem).wait()
                pltpu.sync_copy(rows_v,
                                out_hbm.at[pl.ds(base + ch * GCH, GCH)])

    return g2


def _shared_body(x_ref, uw_ref, ub_ref, dw_ref, db_ref, out_ref):
    xb = x_ref[...].astype(jnp.bfloat16)
    h = lax.dot_general(xb, uw_ref[...], (((1,), (1,)), ((), ())),
                        preferred_element_type=jnp.float32) + ub_ref[...]
    g = _gelu(h).astype(jnp.bfloat16)
    out_ref[...] = lax.dot_general(g, dw_ref[...], (((1,), (1,)), ((), ())),
                                   preferred_element_type=jnp.float32) + db_ref[...]


def _shared(x2d, sh_up_w, sh_up_b, sh_down_w, sh_down_b, T):
    return pl.pallas_call(
        _shared_body,
        grid=(T // TOK,),
        in_specs=[
            pl.BlockSpec((TOK, H), lambda t: (t, 0)),
            pl.BlockSpec((I, H), lambda t: (0, 0)),
            pl.BlockSpec((1, I), lambda t: (0, 0)),
            pl.BlockSpec((H, I), lambda t: (0, 0)),
            pl.BlockSpec((1, H), lambda t: (0, 0)),
        ],
        out_specs=pl.BlockSpec((TOK, H), lambda t: (t, 0)),
        out_shape=jax.ShapeDtypeStruct((T, H), jnp.float32),
    )(x2d, sh_up_w, sh_up_b, sh_down_w, sh_down_b)


def _combine_body(sh_ref, pk_ref, a_ref, b_ref, out_ref):
    pk = pk_ref[...]
    lane = lax.broadcasted_iota(jnp.int32, pk.shape, 1)
    w0 = jnp.sum(jnp.where(lane == 0, pk, 0.0), axis=1, keepdims=True)
    w1 = jnp.sum(jnp.where(lane == 1, pk, 0.0), axis=1, keepdims=True)
    t = sh_ref[...] + w0 * a_ref[...] + w1 * b_ref[...]
    mo = 0.1 * t
    out_ref[...] = mo / (jnp.max(jnp.abs(mo), axis=1, keepdims=True) + 1e-6)


def _combine(sh, packed, a, b, T):
    return pl.pallas_call(
        _combine_body,
        grid=(T // TOK,),
        in_specs=[
            pl.BlockSpec((TOK, H), lambda t: (t, 0)),
            pl.BlockSpec((TOK, LANES), lambda t: (t, 0)),
            pl.BlockSpec((TOK, H), lambda t: (t, 0)),
            pl.BlockSpec((TOK, H), lambda t: (t, 0)),
        ],
        out_specs=pl.BlockSpec((TOK, H), lambda t: (t, 0)),
        out_shape=jax.ShapeDtypeStruct((T, H), jnp.float32),
    )(sh, packed, a, b)


def kernel(x, gate_w, moe_bias, sh_up_w, sh_up_b, sh_down_w, sh_down_b,
           ex_up_w, ex_up_b, ex_down_w, ex_down_b):
    B, S, _ = x.shape
    T = B * S
    P = ((T * K + E * (BT - 1)) + BT - 1) // BT * BT
    x2d = x.reshape(T, H)

    gw_pad = jnp.zeros((LANES, H), jnp.float32).at[:E].set(gate_w)
    gb_pad = jnp.zeros((1, LANES), jnp.float32).at[0, :E].set(moe_bias)

    packed, cnt = _router(x2d, gw_pad, gb_pad, T)
    counts = cnt[0, :E].astype(jnp.int32)
    pad_cnt = ((counts + BT - 1) // BT) * BT
    pad_end = jnp.cumsum(pad_cnt)
    pad_off = (pad_end - pad_cnt).astype(jnp.int32)
    pad_off16 = jnp.zeros((16,), jnp.int32).at[:E].set(pad_off)

    tstart = jnp.arange(P // BT, dtype=jnp.int32) * BT
    te = jnp.minimum(jnp.searchsorted(pad_end, tstart, side="right"),
                     E - 1).astype(jnp.int32)

    e0 = packed[:, 2].astype(jnp.int32)
    e1 = packed[:, 3].astype(jnp.int32)
    r0 = packed[:, 4].astype(jnp.int32)
    r1 = packed[:, 5].astype(jnp.int32)

    sh = _shared(x2d, sh_up_w.astype(jnp.bfloat16), sh_up_b.reshape(1, I),
                 sh_down_w.astype(jnp.bfloat16), sh_down_b.reshape(1, H), T)
    xs = _make_g1(T, P)(pad_off16, e0, r0, e1, r1, x2d)
    ys = _expert_ffn(xs, te, ex_up_w.astype(jnp.bfloat16), ex_up_b,
                     ex_down_w.astype(jnp.bfloat16), ex_down_b, P)
    a, b = _make_g2(T, P)(pad_off16, e0, r0, e1, r1, ys)
    out = _combine(sh, packed, a, b, T)
    return out.reshape(B, S, H)
